# Initial kernel scaffold; baseline (speedup 1.0000x reference)
import jax
import jax.numpy as jnp
from jax.experimental import pallas as pl


def kernel(x, table):
    raise NotImplementedError("write your pallas kernel here")


# speedup vs baseline: 3.5895x; 3.5895x over previous
import functools

import jax
import jax.numpy as jnp
from jax import lax
from jax.experimental import pallas as pl
from jax.experimental.pallas import tpu as pltpu
from jax.experimental.pallas import tpu_sc as plsc

VOCAB = 1000
DIM = 64
BATCH = 4096
SEQ_LEN = 200
TOTAL = BATCH * SEQ_LEN

NUM_CORES = 2
NUM_SUBCORES = 16
NUM_WORKERS = NUM_CORES * NUM_SUBCORES

IDX_PER_ROW = 128
ROWS_PER_STEP = 4
CHUNK = ROWS_PER_STEP * IDX_PER_ROW
PER_WORKER = TOTAL // NUM_WORKERS
STEPS = PER_WORKER // CHUNK


def _emb_body(table_hbm, x_hbm, out_hbm, idx_v, rows_v, sem):
    c = lax.axis_index("c")
    s = lax.axis_index("s")
    wid = s * NUM_CORES + c
    base_row = wid * STEPS * ROWS_PER_STEP

    def step(i, carry):
        r0 = base_row + i * ROWS_PER_STEP
        pltpu.sync_copy(x_hbm.at[pl.ds(r0, ROWS_PER_STEP)], idx_v)
        copies = [
            pltpu.async_copy(
                table_hbm.at[idx_v.at[j]],
                rows_v.at[pl.ds(j * IDX_PER_ROW, IDX_PER_ROW)],
                sem,
            )
            for j in range(ROWS_PER_STEP)
        ]
        for cp in copies:
            cp.wait()
        pltpu.sync_copy(rows_v, out_hbm.at[pl.ds(r0 * IDX_PER_ROW, CHUNK)])
        return carry

    lax.fori_loop(0, STEPS, step, 0)


@functools.partial(
    pl.kernel,
    mesh=plsc.VectorSubcoreMesh(core_axis_name="c", subcore_axis_name="s"),
    out_type=jax.ShapeDtypeStruct((TOTAL, DIM), jnp.float32),
    scratch_types=[
        pltpu.VMEM((ROWS_PER_STEP, IDX_PER_ROW), jnp.int32),
        pltpu.VMEM((CHUNK, DIM), jnp.float32),
        pltpu.SemaphoreType.DMA,
    ],
    compiler_params=pltpu.CompilerParams(use_tc_tiling_on_sc=False),
)
def _emb_call(table_hbm, x_hbm, out_hbm, idx_v, rows_v, sem):
    _emb_body(table_hbm, x_hbm, out_hbm, idx_v, rows_v, sem)


def kernel(x, table):
    xf = x.reshape(TOTAL // IDX_PER_ROW, IDX_PER_ROW).astype(jnp.int32)
    out = _emb_call(table, xf)
    return out.reshape(BATCH, SEQ_LEN, DIM)
